# Initial kernel scaffold; baseline (speedup 1.0000x reference)
import jax
import jax.numpy as jnp
from jax.experimental import pallas as pl


def kernel(x, edge_index, down_W, down_b, pool_w, up_W, up_b):
    raise NotImplementedError("write your pallas kernel here")


# speedup vs baseline: 1.0422x; 1.0422x over previous
import functools
import math

import jax
import jax.numpy as jnp
from jax import lax
from jax.experimental import pallas as pl
from jax.experimental.pallas import tpu as pltpu
from jax.experimental.pallas import tpu_sc as plsc

N_NODES = 10000
N_EDGES = 320000
HID = 128
DEPTH = 4
NUM_CLASSES = 40

NC, NS = 2, 16
NW = NC * NS
EPW = N_EDGES // NW
CH = 80
NCHUNK = EPW // CH


def _pad_rows(n):
    return ((n + 8 * NW - 1) // (8 * NW)) * (8 * NW)


def _mesh():
    return plsc.VectorSubcoreMesh(core_axis_name="c", subcore_axis_name="s")


def sc_degree(dst, np_rows):
    spw = np_rows // NW

    @functools.partial(
        pl.kernel,
        mesh=_mesh(),
        out_type=jax.ShapeDtypeStruct((NC * np_rows, 16), jnp.float32),
        scratch_types=[
            pltpu.VMEM((CH,), jnp.int32),
            pltpu.VMEM((CH, 16), jnp.float32),
            pltpu.VMEM_SHARED((np_rows, 16), jnp.float32),
        ],
    )
    def k(dst_hbm, ones_hbm, zeros_hbm, out_hbm, dstv, onesv, acc_sh):
        cid = lax.axis_index("c")
        wid = lax.axis_index("s") * NC + cid
        woff = wid * spw
        pltpu.sync_copy(zeros_hbm.at[pl.ds(woff, spw)],
                        acc_sh.at[pl.ds(woff, spw)])
        pltpu.sync_copy(ones_hbm, onesv)
        plsc.subcore_barrier()

        def body(i, _):
            base = wid * EPW + i * CH
            pltpu.sync_copy(dst_hbm.at[pl.ds(base, CH)], dstv)
            pltpu.sync_copy(onesv, acc_sh.at[dstv], add=True)
            return _

        lax.fori_loop(0, NCHUNK, body, None)
        plsc.subcore_barrier()
        woff = wid * spw
        pltpu.sync_copy(acc_sh.at[pl.ds(woff, spw)],
                        out_hbm.at[pl.ds(cid * np_rows + woff, spw)])

    out = k(dst,
            jnp.ones((CH, 16), jnp.float32),
            jnp.zeros((np_rows, 16), jnp.float32))
    return out.reshape(NC, np_rows, 16).sum(axis=0)[:, 0]


def sc_message(xw2, src, dst, np_rows):
    spw = np_rows // NW

    @functools.partial(
        pl.kernel,
        mesh=_mesh(),
        out_type=jax.ShapeDtypeStruct((NC * np_rows, HID), jnp.float32),
        scratch_types=[
            pltpu.VMEM((CH,), jnp.int32),
            pltpu.VMEM((CH,), jnp.int32),
            pltpu.VMEM((CH, HID), jnp.float32),
            pltpu.VMEM_SHARED((np_rows, HID), jnp.float32),
            pltpu.SemaphoreType.DMA,
        ],
    )
    def k(xw_hbm, src_hbm, dst_hbm, zeros_hbm, out_hbm,
          srcv, dstv, rows, acc_sh, sem):
        cid = lax.axis_index("c")
        wid = lax.axis_index("s") * NC + cid
        woff = wid * spw
        pltpu.sync_copy(zeros_hbm.at[pl.ds(woff, spw)],
                        acc_sh.at[pl.ds(woff, spw)])
        plsc.subcore_barrier()

        def body(i, _):
            base = wid * EPW + i * CH
            pltpu.sync_copy(src_hbm.at[pl.ds(base, CH)], srcv)
            pltpu.sync_copy(dst_hbm.at[pl.ds(base, CH)], dstv)
            pltpu.async_copy(xw_hbm.at[srcv], rows, sem).wait()
            pltpu.sync_copy(rows, acc_sh.at[dstv], add=True)
            return _

        lax.fori_loop(0, NCHUNK, body, None)
        plsc.subcore_barrier()
        pltpu.sync_copy(acc_sh.at[pl.ds(woff, spw)],
                        out_hbm.at[pl.ds(cid * np_rows + woff, spw)])

    out = k(xw2, src, dst, jnp.zeros((np_rows, HID), jnp.float32))
    return out.reshape(NC, np_rows, HID).sum(axis=0)


def sc_remap(src, dst, remap_map, n_old, dummy):
    @functools.partial(
        pl.kernel,
        mesh=_mesh(),
        out_type=(jax.ShapeDtypeStruct((N_EDGES,), jnp.int32),
                  jax.ShapeDtypeStruct((N_EDGES,), jnp.int32)),
        compiler_params=pltpu.CompilerParams(needs_layout_passes=False),
        scratch_types=[
            pltpu.VMEM((n_old,), jnp.int32),
            pltpu.VMEM((CH,), jnp.int32),
            pltpu.VMEM((CH,), jnp.int32),
            pltpu.VMEM((CH,), jnp.int32),
            pltpu.VMEM((CH,), jnp.int32),
        ],
    )
    def k(src_hbm, dst_hbm, map_hbm, ns_hbm, nd_hbm,
          mapv, srcv, dstv, outs, outd):
        cid = lax.axis_index("c")
        wid = lax.axis_index("s") * NC + cid
        pltpu.sync_copy(map_hbm, mapv)

        def body(i, _):
            base = wid * EPW + i * CH
            pltpu.sync_copy(src_hbm.at[pl.ds(base, CH)], srcv)
            pltpu.sync_copy(dst_hbm.at[pl.ds(base, CH)], dstv)
            for j in range(CH // 16):
                sl = pl.ds(j * 16, 16)
                gs = plsc.load_gather(mapv, [srcv[sl]])
                gd = plsc.load_gather(mapv, [dstv[sl]])
                dead = (gs == dummy) | (gd == dummy)
                outs[sl] = jnp.where(dead, dummy, gs)
                outd[sl] = jnp.where(dead, dummy, gd)
            pltpu.sync_copy(outs, ns_hbm.at[pl.ds(base, CH)])
            pltpu.sync_copy(outd, nd_hbm.at[pl.ds(base, CH)])
            return _

        lax.fori_loop(0, NCHUNK, body, None)

    return k(src, dst, remap_map)


def _mm_body(x_ref, w_ref, o_ref):
    o_ref[:] = jnp.dot(x_ref[:], w_ref[:], preferred_element_type=jnp.float32)


def tc_matmul(x, w):
    return pl.pallas_call(
        _mm_body,
        out_shape=jax.ShapeDtypeStruct((x.shape[0], w.shape[1]), jnp.float32),
    )(x, w)


def _gcn_conv(x, src, dst, dinv, W, b):
    n = x.shape[0]
    np_rows = _pad_rows(n + 1)
    xw = tc_matmul(x, W)
    xw2 = xw * dinv[:, None]
    xw2 = jnp.pad(xw2, ((0, np_rows - n), (0, 0)))
    s = sc_message(xw2, src, dst, np_rows)[:n]
    return dinv[:, None] * s + 2.0 * (dinv ** 2)[:, None] * xw + b


def _degree_inv(dst, n):
    np_rows = _pad_rows(n + 1)
    hist = sc_degree(dst, np_rows)[:n]
    return lax.rsqrt(hist + 2.0)


def kernel(x, edge_index, down_W, down_b, pool_w, up_W, up_b):
    src = edge_index[0].astype(jnp.int32)
    dst = edge_index[1].astype(jnp.int32)

    ns = [N_NODES]
    ratios = [2000.0 / N_NODES, 0.5, 0.5, 0.4]
    for r in ratios:
        ns.append(int(math.ceil(r * ns[-1])))

    dinv = _degree_inv(dst, ns[0])
    h = jax.nn.relu(_gcn_conv(x, src, dst, dinv, down_W[0], down_b[0]))

    xs, srcs, dsts, dinvs, perms = [h], [src], [dst], [dinv], []
    for i in range(1, DEPTH + 1):
        n_old, k = ns[i - 1], ns[i]
        p = pool_w[i - 1]
        p8 = jnp.tile(p[:, None], (1, 8))
        score = jnp.tanh(tc_matmul(h, p8)[:, 0] / jnp.linalg.norm(p))
        topv, perm = lax.top_k(score, k)
        hk = h[perm] * topv[:, None]
        remap_map = jnp.full((n_old,), k, jnp.int32).at[perm].set(
            jnp.arange(k, dtype=jnp.int32))
        src, dst = sc_remap(src, dst, remap_map, n_old, k)
        dinv = _degree_inv(dst, k)
        h = jax.nn.relu(_gcn_conv(hk, src, dst, dinv, down_W[i], down_b[i]))
        if i < DEPTH:
            xs.append(h)
            srcs.append(src)
            dsts.append(dst)
            dinvs.append(dinv)
        perms.append(perm)

    for i in range(DEPTH):
        j = DEPTH - 1 - i
        res = xs[j]
        hin = res.at[perms[j]].add(h)
        W, b = up_W[i], up_b[i]
        if W.shape[1] < HID:
            W = jnp.pad(W, ((0, 0), (0, HID - W.shape[1])))
            b = jnp.pad(b, (0, HID - b.shape[0]))
        h = _gcn_conv(hin, srcs[j], dsts[j], dinvs[j], W, b)
        if i < DEPTH - 1:
            h = jax.nn.relu(h)

    out = h[:, :NUM_CLASSES]
    return jax.nn.log_softmax(out, axis=1)


# speedup vs baseline: 1.0517x; 1.0092x over previous
import functools
import math

import jax
import jax.numpy as jnp
from jax import lax
from jax.experimental import pallas as pl
from jax.experimental.pallas import tpu as pltpu
from jax.experimental.pallas import tpu_sc as plsc

N_NODES = 10000
N_EDGES = 320000
HID = 128
DEPTH = 4
NUM_CLASSES = 40

NC, NS = 2, 16
NW = NC * NS
EPW = N_EDGES // NW
CH = 40
NCHUNK = EPW // CH
NB = 5


def _pad_rows(n):
    return ((n + 8 * NW - 1) // (8 * NW)) * (8 * NW)


def _mesh():
    return plsc.VectorSubcoreMesh(core_axis_name="c", subcore_axis_name="s")


def sc_degree(dst3, np_rows):
    spw = np_rows // NW

    @functools.partial(
        pl.kernel,
        mesh=_mesh(),
        out_type=jax.ShapeDtypeStruct((NC * np_rows, 16), jnp.float32),
        scratch_types=[
            pltpu.VMEM((NB, CH), jnp.int32),
            pltpu.VMEM((CH, 16), jnp.float32),
            pltpu.VMEM_SHARED((np_rows, 16), jnp.float32),
            pltpu.SemaphoreType.DMA,
        ],
    )
    def k(dst_hbm, ones_hbm, zeros_hbm, out_hbm, dstv, onesv, acc_sh, sem):
        cid = lax.axis_index("c")
        wid = lax.axis_index("s") * NC + cid
        woff = wid * spw
        pltpu.sync_copy(zeros_hbm.at[pl.ds(woff, spw)],
                        acc_sh.at[pl.ds(woff, spw)])
        pltpu.sync_copy(ones_hbm, onesv)
        plsc.subcore_barrier()

        def body(i, _):
            ihs = []
            for b in range(NB):
                base = wid * EPW + (i * NB + b) * CH
                ihs.append(pltpu.async_copy(
                    dst_hbm.at[pl.ds(base, CH)], dstv.at[b], sem))
            for h in ihs:
                h.wait()
            hs = []
            for b in range(NB):
                hs.append(pltpu.async_copy(
                    onesv, acc_sh.at[dstv.at[b]], sem, add=True))
            for h in hs:
                h.wait()
            return _

        lax.fori_loop(0, NCHUNK // NB, body, None)
        plsc.subcore_barrier()
        pltpu.sync_copy(acc_sh.at[pl.ds(woff, spw)],
                        out_hbm.at[pl.ds(cid * np_rows + woff, spw)])

    out = k(dst3,
            jnp.ones((CH, 16), jnp.float32),
            jnp.zeros((np_rows, 16), jnp.float32))
    return out.reshape(NC, np_rows, 16).sum(axis=0)[:, 0]


def sc_message(xw2, src3, dst3, np_rows):
    spw = np_rows // NW

    @functools.partial(
        pl.kernel,
        mesh=_mesh(),
        out_type=jax.ShapeDtypeStruct((NC * np_rows, HID), jnp.float32),
        scratch_types=[
            pltpu.VMEM((NB, CH), jnp.int32),
            pltpu.VMEM((NB, CH), jnp.int32),
            pltpu.VMEM((NB, CH, HID), jnp.float32),
            pltpu.VMEM_SHARED((np_rows, HID), jnp.float32),
            pltpu.SemaphoreType.DMA,
            pltpu.SemaphoreType.DMA,
        ],
    )
    def k(xw_hbm, src_hbm, dst_hbm, zeros_hbm, out_hbm,
          srcv, dstv, rows, acc_sh, gsem, ssem):
        cid = lax.axis_index("c")
        wid = lax.axis_index("s") * NC + cid
        woff = wid * spw
        pltpu.sync_copy(zeros_hbm.at[pl.ds(woff, spw)],
                        acc_sh.at[pl.ds(woff, spw)])
        plsc.subcore_barrier()

        def body(i, _):
            ihs = []
            for b in range(NB):
                base = wid * EPW + (i * NB + b) * CH
                ihs.append(pltpu.async_copy(
                    src_hbm.at[pl.ds(base, CH)], srcv.at[b], gsem))
                ihs.append(pltpu.async_copy(
                    dst_hbm.at[pl.ds(base, CH)], dstv.at[b], gsem))
            for h in ihs:
                h.wait()
            ghs = []
            for b in range(NB):
                ghs.append(pltpu.async_copy(
                    xw_hbm.at[srcv.at[b]], rows.at[b], gsem))
            shs = []
            for b in range(NB):
                ghs[b].wait()
                shs.append(pltpu.async_copy(
                    rows.at[b], acc_sh.at[dstv.at[b]], ssem,
                    add=True))
            for h in shs:
                h.wait()
            return _

        lax.fori_loop(0, NCHUNK // NB, body, None)
        plsc.subcore_barrier()
        pltpu.sync_copy(acc_sh.at[pl.ds(woff, spw)],
                        out_hbm.at[pl.ds(cid * np_rows + woff, spw)])

    out = k(xw2, src3, dst3, jnp.zeros((np_rows, HID), jnp.float32))
    return out.reshape(NC, np_rows, HID).sum(axis=0)


def sc_remap(src, dst, remap_map, n_old, dummy):
    @functools.partial(
        pl.kernel,
        mesh=_mesh(),
        out_type=(jax.ShapeDtypeStruct((N_EDGES,), jnp.int32),
                  jax.ShapeDtypeStruct((N_EDGES,), jnp.int32)),
        compiler_params=pltpu.CompilerParams(needs_layout_passes=False),
        scratch_types=[
            pltpu.VMEM((n_old,), jnp.int32),
            pltpu.VMEM((EPW,), jnp.int32),
            pltpu.VMEM((EPW,), jnp.int32),
            pltpu.VMEM((EPW,), jnp.int32),
            pltpu.VMEM((EPW,), jnp.int32),
        ],
    )
    def k(src_hbm, dst_hbm, map_hbm, ns_hbm, nd_hbm,
          mapv, srcv, dstv, outs, outd):
        cid = lax.axis_index("c")
        wid = lax.axis_index("s") * NC + cid
        base = wid * EPW
        pltpu.sync_copy(map_hbm, mapv)
        pltpu.sync_copy(src_hbm.at[pl.ds(base, EPW)], srcv)
        pltpu.sync_copy(dst_hbm.at[pl.ds(base, EPW)], dstv)

        def body(q, _):
            sl = pl.ds(q * 16, 16)
            gs = plsc.load_gather(mapv, [srcv[sl]])
            gd = plsc.load_gather(mapv, [dstv[sl]])
            dead = (gs == dummy) | (gd == dummy)
            outs[sl] = jnp.where(dead, dummy, gs)
            outd[sl] = jnp.where(dead, dummy, gd)
            return _

        lax.fori_loop(0, EPW // 16, body, None)
        pltpu.sync_copy(outs, ns_hbm.at[pl.ds(base, EPW)])
        pltpu.sync_copy(outd, nd_hbm.at[pl.ds(base, EPW)])

    return k(src, dst, remap_map)


def _mm_body(x_ref, w_ref, o_ref):
    o_ref[:] = jnp.dot(x_ref[:], w_ref[:], preferred_element_type=jnp.float32)


def tc_matmul(x, w):
    return pl.pallas_call(
        _mm_body,
        out_shape=jax.ShapeDtypeStruct((x.shape[0], w.shape[1]), jnp.float32),
    )(x, w)


def _gcn_conv(x, src, dst, dinv, W, b):
    n = x.shape[0]
    np_rows = _pad_rows(n + 1)
    xw = tc_matmul(x, W)
    xw2 = xw * dinv[:, None]
    xw2 = jnp.pad(xw2, ((0, np_rows - n), (0, 0)))
    s = sc_message(xw2, src, dst, np_rows)[:n]
    return dinv[:, None] * s + 2.0 * (dinv ** 2)[:, None] * xw + b


def _degree_inv(dst, n):
    np_rows = _pad_rows(n + 1)
    hist = sc_degree(dst, np_rows)[:n]
    return lax.rsqrt(hist + 2.0)


def kernel(x, edge_index, down_W, down_b, pool_w, up_W, up_b):
    src = edge_index[0].astype(jnp.int32)
    dst = edge_index[1].astype(jnp.int32)

    ns = [N_NODES]
    ratios = [2000.0 / N_NODES, 0.5, 0.5, 0.4]
    for r in ratios:
        ns.append(int(math.ceil(r * ns[-1])))

    dinv = _degree_inv(dst, ns[0])
    h = jax.nn.relu(_gcn_conv(x, src, dst, dinv, down_W[0], down_b[0]))

    xs, srcs, dsts, dinvs, perms = [h], [src], [dst], [dinv], []
    for i in range(1, DEPTH + 1):
        n_old, k = ns[i - 1], ns[i]
        p = pool_w[i - 1]
        p8 = jnp.tile(p[:, None], (1, 8))
        score = jnp.tanh(tc_matmul(h, p8)[:, 0] / jnp.linalg.norm(p))
        topv, perm = lax.top_k(score, k)
        hk = h[perm] * topv[:, None]
        remap_map = jnp.full((n_old,), k, jnp.int32).at[perm].set(
            jnp.arange(k, dtype=jnp.int32))
        src, dst = sc_remap(src, dst, remap_map, n_old, k)
        dinv = _degree_inv(dst, k)
        h = jax.nn.relu(_gcn_conv(hk, src, dst, dinv, down_W[i], down_b[i]))
        if i < DEPTH:
            xs.append(h)
            srcs.append(src)
            dsts.append(dst)
            dinvs.append(dinv)
        perms.append(perm)

    for i in range(DEPTH):
        j = DEPTH - 1 - i
        res = xs[j]
        hin = res.at[perms[j]].add(h)
        W, b = up_W[i], up_b[i]
        if W.shape[1] < HID:
            W = jnp.pad(W, ((0, 0), (0, HID - W.shape[1])))
            b = jnp.pad(b, (0, HID - b.shape[0]))
        h = _gcn_conv(hin, srcs[j], dsts[j], dinvs[j], W, b)
        if i < DEPTH - 1:
            h = jax.nn.relu(h)

    out = h[:, :NUM_CLASSES]
    return jax.nn.log_softmax(out, axis=1)
